# baseline (speedup 1.0000x reference)
import functools

import jax
import jax.numpy as jnp
from jax import lax
from jax.experimental import pallas as pl
from jax.experimental.pallas import tpu as pltpu
from jax.experimental.pallas import tpu_sc as plsc

HD = 128
NC, NS, L = 2, 16, 16
NW = NC * NS
E = 64000
EC = 80
ECH_TOT = E // EC
ECH = ECH_TOT // NW
ET = E // NW
P10 = 10240
P5 = 5120
BM = 512

_f32 = jnp.float32
_i32 = jnp.int32


def _pad_rows(x, n):
    return jnp.pad(x, ((0, n - x.shape[0]),) + ((0, 0),) * (x.ndim - 1))


def _sc_gather_call(tables, idxs):
    mesh = plsc.VectorSubcoreMesh(core_axis_name="c", subcore_axis_name="s", num_cores=NC, num_subcores=NS)
    specs = [(t.shape, int(i.shape[0])) for t, i in zip(tables, idxs)]
    maxc = max(b // (NW * EC) for _, b in specs)

    out_type = [jax.ShapeDtypeStruct((b, HD), _f32) for _, b in specs]
    nch_tot = sum(b // (NW * EC) for _, b in specs)
    del maxc
    scratch = [pltpu.VMEM((nch_tot, EC), _i32),
               pltpu.VMEM((2, EC, HD), _f32),
               pltpu.SemaphoreType.DMA, pltpu.SemaphoreType.DMA,
               pltpu.SemaphoreType.DMA]

    chunks = []
    for g, (_, b) in enumerate(specs):
        for k in range(b // (NW * EC)):
            chunks.append((g, k))
    ncht = len(chunks)

    def body(*refs):
        n = len(specs)
        tabs = refs[:n]
        idr = refs[n:2 * n]
        outs = refs[2 * n:3 * n]
        idx_v, rows_v, si, sg, so = refs[3 * n:]
        wid = lax.axis_index("c") * NS + lax.axis_index("s")

        def off_of(g, k):
            nch = specs[g][1] // (NW * EC)
            return wid * nch * EC + k * EC

        idd = []
        for i, (g, k) in enumerate(chunks):
            idd.append(pltpu.async_copy(
                idr[g].at[pl.ds(off_of(g, k), EC)], idx_v.at[i], si))
        for d in idd:
            d.wait()
        gd = [None] * ncht
        od = [None] * ncht
        g0, k0 = chunks[0]
        gd[0] = pltpu.async_copy(tabs[g0].at[idx_v.at[0]], rows_v.at[0], sg)
        for i, (g, k) in enumerate(chunks):
            b = i % 2
            gd[i].wait()
            if i + 1 < ncht:
                if i >= 1:
                    od[i - 1].wait()
                g1, k1 = chunks[i + 1]
                gd[i + 1] = pltpu.async_copy(
                    tabs[g1].at[idx_v.at[i + 1]], rows_v.at[1 - b], sg)
            od[i] = pltpu.async_copy(
                rows_v.at[b], outs[g].at[pl.ds(off_of(g, k), EC)], so)
        od[ncht - 2].wait()
        od[ncht - 1].wait()

    fn = pl.kernel(body, out_type=out_type, mesh=mesh, scratch_types=scratch, compiler_params=pltpu.CompilerParams(needs_layout_passes=False))
    return fn(*tables, *idxs)


def _sc_att_softmax_multi(rels):
    nrel = len(rels)
    n_src_max = max(r[2].shape[0] for r in rels)
    mesh = plsc.VectorSubcoreMesh(core_axis_name="c", subcore_axis_name="s", num_cores=NC, num_subcores=NS)
    out_type = ([jax.ShapeDtypeStruct((NW, ECH, EC), _f32)] * nrel +
                [jax.ShapeDtypeStruct((NC, P10 // L, L), _f32)] * nrel +
                [jax.ShapeDtypeStruct((NC, L), _f32)] * nrel)
    scratch = [
        pltpu.VMEM((ECH, EC), _i32),
        pltpu.VMEM((ECH, EC), _i32),
        pltpu.VMEM((n_src_max,), _f32),
        pltpu.VMEM((P10,), _f32),
        pltpu.VMEM((ECH, EC), _f32),
        pltpu.VMEM((2, EC, L), _f32),
        pltpu.VMEM((2, EC), _i32),
        pltpu.VMEM((P10 // L // NS, L), _f32),
        pltpu.VMEM((NS, L), _f32),
        pltpu.VMEM((L,), _f32),
        pltpu.VMEM_SHARED((P10 // L, L), _f32),
        pltpu.VMEM_SHARED((NS, L), _f32),
        pltpu.SemaphoreType.DMA,
    ]

    def body(*refs):
        ins = refs[:4 * nrel]
        outE = refs[4 * nrel:5 * nrel]
        outS = refs[5 * nrel:6 * nrel]
        outM = refs[6 * nrel:7 * nrel]
        (src2, dst2, ssrc_v, sdst_v, abuf, oh, ohx, sbuf,
         mall, mbuf, S_sh2, M_sh, so) = refs[7 * nrel:]
        cid = lax.axis_index("c")
        sid = lax.axis_index("s")
        wid = cid * NS + sid
        zv = jnp.zeros((L,), _f32)
        nsr = P10 // L // NS
        iv = lax.iota(_i32, L)

        def zsb(i, _):
            sbuf[i, :] = zv
            return 0
        lax.fori_loop(0, nsr, zsb, 0)

        for rel in range(nrel):
            src_h, dst_h, ssrc_h, sdst_h = ins[4 * rel:4 * rel + 4]
            nsp = ssrc_h.shape[0]

            pltpu.sync_copy(sbuf, S_sh2.at[pl.ds(sid * nsr, nsr)])

            pltpu.sync_copy(src_h.at[wid], src2)
            pltpu.sync_copy(dst_h.at[wid], dst2)
            pltpu.sync_copy(ssrc_h, ssrc_v.at[pl.ds(0, nsp)])
            pltpu.sync_copy(sdst_h, sdst_v)

            def apass(ch, mv):
                for v in range(EC // L):
                    s16 = src2[ch, pl.ds(v * L, L)]
                    d16 = dst2[ch, pl.ds(v * L, L)]
                    a = plsc.load_gather(ssrc_v, [s16])
                    b = plsc.load_gather(sdst_v, [d16])
                    al = a + b
                    al = jnp.where(al > 0, al, 0.2 * al)
                    abuf[ch, pl.ds(v * L, L)] = al
                    mv = jnp.maximum(mv, al)
                return mv
            mv = lax.fori_loop(0, ECH, apass, jnp.full((L,), -1e30, _f32))

            mbuf[...] = mv
            pltpu.sync_copy(mbuf, M_sh.at[sid])
            plsc.subcore_barrier()
            pltpu.sync_copy(M_sh, mall)
            mm = mall[0, :]
            for i in range(1, NS):
                mm = jnp.maximum(mm, mall[i, :])
            M = jnp.max(mm)

            sdd = [None] * ECH
            for ch in range(ECH):
                b = ch % 2
                if ch >= 2:
                    sdd[ch - 2].wait()

                def zoh(r, _2, b=b):
                    oh[b, r, :] = zv
                    return 0
                lax.fori_loop(0, EC, zoh, 0)
                for v in range(EC // L):
                    d16 = dst2[ch, pl.ds(v * L, L)]
                    al = abuf[ch, pl.ds(v * L, L)]
                    e = jnp.exp(al - M)
                    abuf[ch, pl.ds(v * L, L)] = e
                    plsc.store_scatter(
                        oh.at[b], [iv + v * L,
                                   jnp.bitwise_and(d16, L - 1)], e)
                    ohx[b, pl.ds(v * L, L)] = lax.shift_right_logical(d16, 4)
                sdd[ch] = pltpu.async_copy(oh.at[b], S_sh2.at[ohx.at[b]],
                                           so, add=True)
            sdd[ECH - 2].wait()
            sdd[ECH - 1].wait()

            pltpu.sync_copy(abuf, outE[rel].at[wid])
            plsc.subcore_barrier()

            pltpu.sync_copy(S_sh2.at[pl.ds(sid * nsr, nsr)],
                            outS[rel].at[cid, pl.ds(sid * nsr, nsr)])

            mbuf[...] = jnp.full((L,), M, _f32)

            @pl.when(sid == 0)
            def _():
                pltpu.sync_copy(mbuf, outM[rel].at[cid])

    fn = pl.kernel(body, out_type=out_type, mesh=mesh, scratch_types=scratch, compiler_params=pltpu.CompilerParams(needs_layout_passes=False))
    outs = fn(*[x for r in rels for x in r])
    return [(outs[i], outs[nrel + i], outs[2 * nrel + i])
            for i in range(nrel)]


def _sc_att_aggregate_multi(rels):
    nrel = len(rels)
    mesh = plsc.VectorSubcoreMesh(core_axis_name="c", subcore_axis_name="s", num_cores=NC, num_subcores=NS)
    out_type = [jax.ShapeDtypeStruct((NC, P10, HD), _f32)] * nrel
    scratch = [
        pltpu.VMEM((ECH, EC), _i32),
        pltpu.VMEM((ECH, EC), _i32),
        pltpu.VMEM((ECH, EC), _f32),
        pltpu.VMEM((2, EC, HD), _f32),
        pltpu.VMEM_SHARED((P10, HD), _f32),
        pltpu.SemaphoreType.DMA,
        pltpu.SemaphoreType.DMA,
    ]

    def body(*refs):
        ins = refs[:4 * nrel]
        outP = refs[4 * nrel:5 * nrel]
        src2, dst2, ebuf_v, rows, P_sh, sg, ss = refs[5 * nrel:]
        cid = lax.axis_index("c")
        sid = lax.axis_index("s")
        wid = cid * NS + sid
        zv = jnp.zeros((L,), _f32)
        nrow_t = P10 // NS

        for rel in range(nrel):
            src_h, dst_h, e_h, h_h = ins[4 * rel:4 * rel + 4]

            def zrow(i, _):
                for j in range(HD // L):
                    rows[0, i, pl.ds(j * L, L)] = zv
                return 0
            lax.fori_loop(0, EC, zrow, 0)
            for k in range(nrow_t // EC):
                pltpu.sync_copy(rows.at[0],
                                P_sh.at[pl.ds(sid * nrow_t + k * EC, EC)])

            pltpu.sync_copy(src_h.at[wid], src2)
            pltpu.sync_copy(dst_h.at[wid], dst2)
            pltpu.sync_copy(e_h.at[wid], ebuf_v)

            plsc.subcore_barrier()

            gd = [None] * ECH
            sd = [None] * ECH
            gd[0] = pltpu.async_copy(h_h.at[src2.at[0]], rows.at[0], sg)
            for ch in range(ECH):
                b = ch % 2
                gd[ch].wait()
                if ch >= 1:
                    sd[ch - 1].wait()
                if ch + 1 < ECH:
                    gd[ch + 1] = pltpu.async_copy(
                        h_h.at[src2.at[ch + 1]], rows.at[1 - b], sg)

                def scale(r, _2, ch=ch, b=b):
                    ev = plsc.load_gather(
                        ebuf_v,
                        [jnp.full((L,), ch, _i32), jnp.full((L,), r, _i32)])
                    for j in range(HD // L):
                        rows[b, r, pl.ds(j * L, L)] = (
                            rows[b, r, pl.ds(j * L, L)] * ev)
                    return 0
                lax.fori_loop(0, EC, scale, 0)
                sd[ch] = pltpu.async_copy(rows.at[b], P_sh.at[dst2.at[ch]],
                                          ss, add=True)
            sd[ECH - 1].wait()

            plsc.subcore_barrier()
            pltpu.sync_copy(P_sh.at[pl.ds(sid * nrow_t, nrow_t)],
                            outP[rel].at[cid, pl.ds(sid * nrow_t, nrow_t)])

    fn = pl.kernel(body, out_type=out_type, mesh=mesh, scratch_types=scratch, compiler_params=pltpu.CompilerParams(needs_layout_passes=False))
    outs = fn(*[x for r in rels for x in r])
    return list(outs) if nrel > 1 else [outs[0] if isinstance(outs, (list, tuple)) else outs]


def _sc_edge_att(src_r, dst_r, h_src, s_src, s_dst):
    [(ebuf, S, M)] = _sc_att_softmax_multi([(src_r, dst_r, s_src, s_dst)])
    [P] = _sc_att_aggregate_multi([(src_r, dst_r, ebuf, h_src)])
    return P, S, M


def _edge_att_combine(parts, n_dst):
    ws = []
    for (Pp, Sr, Mm) in parts:
        m0 = Mm[0, 0]
        m1 = Mm[1, 0]
        mg = jnp.maximum(m0, m1)
        ws.append(jnp.stack([jnp.exp(m0 - mg), jnp.exp(m1 - mg)]))
    w = jnp.concatenate(ws)
    arrs = []
    for (Pp, Sr, Mm) in parts:
        arrs.extend([Pp, Sr.reshape(NC, P10 // HD, HD)])
    return _comb_call(arrs, w, len(parts))


def _mm_body(xr, wr, br, ar, yr, scr, str_, *, n_real, renorm):
    i = pl.program_id(0)
    x = xr[...]
    if renorm:
        nrm = jnp.sqrt(jnp.sum(x * x, axis=1, keepdims=True))
        x = x * jnp.minimum(1.0, 1.0 / (nrm + 1e-12))
    y = jnp.dot(x, wr[...], preferred_element_type=_f32) + br[...]
    rid = lax.broadcasted_iota(_i32, y.shape, 0) + i * BM
    y = jnp.where(rid < n_real, y, 0.0)
    yr[...] = y
    scr[...] = jnp.dot(y, ar[...], preferred_element_type=_f32)
    s1 = jnp.sum(y, axis=0, keepdims=True)
    s2 = jnp.sum(y * y, axis=0, keepdims=True)
    str_[...] = jnp.concatenate([s1, s2], axis=0)[None]


def _mm_call(x, W, b, avecs, n_real, renorm=False):
    np_ = x.shape[0]
    nblk = np_ // BM
    A = jnp.zeros((HD, 8), _f32)
    for k, v in enumerate(avecs):
        A = A.at[:, k].set(v)
    f = pl.pallas_call(
        functools.partial(_mm_body, n_real=n_real, renorm=renorm),
        grid=(nblk,),
        in_specs=[pl.BlockSpec((BM, HD), lambda i: (i, 0)),
                  pl.BlockSpec((HD, HD), lambda i: (0, 0)),
                  pl.BlockSpec((1, HD), lambda i: (0, 0)),
                  pl.BlockSpec((HD, 8), lambda i: (0, 0))],
        out_specs=[pl.BlockSpec((BM, HD), lambda i: (i, 0)),
                   pl.BlockSpec((BM, 8), lambda i: (i, 0)),
                   pl.BlockSpec((1, 2, HD), lambda i: (i, 0, 0))],
        out_shape=[jax.ShapeDtypeStruct((np_, HD), _f32),
                   jax.ShapeDtypeStruct((np_, 8), _f32),
                   jax.ShapeDtypeStruct((nblk, 2, HD), _f32)],
    )
    return f(x, W, b.reshape(1, HD), A)


def _enc_body(xr, w1r, b1r, w2r, b2r, zr, *, n_real):
    i = pl.program_id(0)
    h = jnp.maximum(jnp.dot(xr[...], w1r[...], preferred_element_type=_f32)
                    + b1r[...], 0.0)
    z = jnp.dot(h, w2r[...], preferred_element_type=_f32) + b2r[...]
    nrm = jnp.sqrt(jnp.sum(z * z, axis=1, keepdims=True))
    z = z / (nrm + 1e-12)
    rid = lax.broadcasted_iota(_i32, z.shape, 0) + i * BM
    zr[...] = jnp.where(rid < n_real, z, 0.0)


def _enc_call(x, W1, b1, W2, b2, n_real):
    np_ = x.shape[0]
    f = pl.pallas_call(
        functools.partial(_enc_body, n_real=n_real),
        grid=(np_ // BM,),
        in_specs=[pl.BlockSpec((BM, HD), lambda i: (i, 0)),
                  pl.BlockSpec((HD, HD), lambda i: (0, 0)),
                  pl.BlockSpec((1, HD), lambda i: (0, 0)),
                  pl.BlockSpec((HD, HD), lambda i: (0, 0)),
                  pl.BlockSpec((1, HD), lambda i: (0, 0))],
        out_specs=pl.BlockSpec((BM, HD), lambda i: (i, 0)),
        out_shape=jax.ShapeDtypeStruct((np_, HD), _f32),
    )
    return f(x, W1, b1.reshape(1, HD), W2, b2.reshape(1, HD))


def _flash_body(z1r, z2r, outr, *, n_real, np_):
    i = pl.program_id(0)
    z1b = z1r[...]
    CB = 2048

    def col(c, acc):
        z2c = z2r[pl.ds(c * CB, CB), :]
        s = lax.dot_general(z1b, z2c, (((1,), (1,)), ((), ()))) * 2.0
        cidx = lax.broadcasted_iota(_i32, s.shape, 1) + c * CB
        e = jnp.where(cidx < n_real, jnp.exp(s), 0.0)
        return acc + jnp.sum(e, axis=1, keepdims=True)
    acc = lax.fori_loop(0, np_ // CB, col, jnp.zeros((BM, 1), _f32))

    z2b = z2r[pl.ds(i * BM, BM), :]
    dg = jnp.sum(z1b * z2b, axis=1, keepdims=True) * 2.0
    rid = lax.broadcasted_iota(_i32, (BM, 1), 0) + i * BM
    contrib = jnp.where(rid < n_real, jnp.log(acc) - dg, 0.0)

    @pl.when(i == 0)
    def _():
        outr[...] = jnp.zeros((1, 1), _f32)
    outr[...] += jnp.sum(contrib).reshape(1, 1)


def _flash_call(z1, z2, n_real):
    np_ = z1.shape[0]
    f = pl.pallas_call(
        functools.partial(_flash_body, n_real=n_real, np_=np_),
        grid=(np_ // BM,),
        in_specs=[pl.BlockSpec((BM, HD), lambda i: (i, 0)),
                  pl.BlockSpec((np_, HD), lambda i: (0, 0))],
        out_specs=pl.BlockSpec((1, 1), lambda i: (0, 0)),
        out_shape=jax.ShapeDtypeStruct((1, 1), _f32),
    )
    return f(z1, z2)[0, 0] / n_real


def _comb_body(*refs, nrel, scale):
    wr = refs[0]
    yr, str_ = refs[-2], refs[-1]
    acc = None
    i = pl.program_id(0)
    nsb = BM // HD
    lane = lax.broadcasted_iota(_i32, (BM, HD), 1)
    rowm = jnp.bitwise_and(lax.broadcasted_iota(_i32, (BM, HD), 0), HD - 1)
    emask = jnp.where(lane == rowm, 1.0, 0.0)
    for r in range(nrel):
        Pr = refs[1 + 2 * r][...]
        Sr = refs[2 + 2 * r][:, pl.ds(i * nsb, nsb), :]
        num = wr[2 * r] * Pr[0] + wr[2 * r + 1] * Pr[1]
        den = wr[2 * r] * Sr[0] + wr[2 * r + 1] * Sr[1] + 1e-16
        dfull = jnp.reshape(
            lax.broadcast_in_dim(den, (nsb, HD, HD), (0, 2)), (BM, HD))
        dcol = jnp.sum(dfull * emask, axis=1, keepdims=True)
        t = num / dcol
        acc = t if acc is None else acc + t
    y = jnp.maximum(acc * scale, 0.0)
    yr[...] = y
    s1 = jnp.sum(y, axis=0, keepdims=True)
    s2 = jnp.sum(y * y, axis=0, keepdims=True)
    str_[...] = jnp.concatenate([s1, s2], axis=0)[None]


def _comb_call(arrs, w, nrel):
    nblk = P10 // BM
    scale = 1.0 / nrel
    specs = []
    for k in range(2 * nrel):
        if k % 2 == 0:
            specs.append(pl.BlockSpec((NC, BM, HD), lambda i: (0, i, 0)))
        else:
            specs.append(pl.BlockSpec((NC, P10 // HD, HD),
                                      lambda i: (0, 0, 0)))
    f = pl.pallas_call(
        functools.partial(_comb_body, nrel=nrel, scale=scale),
        grid=(nblk,),
        in_specs=[pl.BlockSpec(memory_space=pltpu.SMEM)] + specs,
        out_specs=[pl.BlockSpec((BM, HD), lambda i: (i, 0)),
                   pl.BlockSpec((1, 2, HD), lambda i: (i, 0, 0))],
        out_shape=[jax.ShapeDtypeStruct((P10, HD), _f32),
                   jax.ShapeDtypeStruct((nblk, 2, HD), _f32)],
    )
    return f(w, *arrs)


def _bn_body(xr, scr, shr, yr):
    yr[...] = xr[...] * scr[...] + shr[...]


def _bn_call(x, sc, sh):
    np_ = x.shape[0]
    f = pl.pallas_call(
        _bn_body,
        grid=(np_ // BM,),
        in_specs=[pl.BlockSpec((BM, HD), lambda i: (i, 0)),
                  pl.BlockSpec((1, HD), lambda i: (0, 0)),
                  pl.BlockSpec((1, HD), lambda i: (0, 0))],
        out_specs=pl.BlockSpec((BM, HD), lambda i: (i, 0)),
        out_shape=jax.ShapeDtypeStruct((np_, HD), _f32),
    )
    return f(x, sc.reshape(1, HD), sh.reshape(1, HD))


def _bn_from_stats(st, n_real, gamma, beta):
    s = jnp.sum(st, axis=0)
    mean = s[0] / n_real
    var = s[1] / n_real - mean * mean
    scale = gamma / jnp.sqrt(var + 1e-5)
    shift = beta - mean * scale
    return scale, shift


def kernel(user_ids, image_ids, ingredient_ids, taste_ids, nutrient, item_x,
           ei_part_of, ei_taste_item, ei_intention_item, ei_image_item,
           ei_user_buys, ei_item_bought, user_table, visual_table,
           caption_table, ing_table, direction_table, W_nut, b_nut, W1, b1,
           W2, b2, Wp_ing, Wp_taste, a1s, a1d, Wn_user, Wn_item, Wn_ing,
           Wn_taste, Wn_int, Wn_img, a_ti_s, a_ti_d, a_ii_s, a_ii_d, a_mi_s,
           a_mi_d, a_ub_s, a_ub_d, a_ib_s, a_ib_d, gamma, beta):
    NU, NIT, NING, NT = 10000, 10000, 5000, 10000
    NINT, NIMG = 10000, 10000

    pad_i = lambda ids, n: jnp.pad(ids.astype(_i32), (0, n - ids.shape[0]))
    ux, visual_x, caption_x, ingredient_x, cooking_x = _sc_gather_call(
        [user_table, visual_table, caption_table, ing_table, direction_table],
        [pad_i(user_ids, P10), pad_i(image_ids, P10), pad_i(image_ids, P10),
         pad_i(ingredient_ids, P5), pad_i(taste_ids, P10)])

    nut_p = _pad_rows(jnp.pad(nutrient, ((0, 0), (0, HD - 20))), P10)
    Wnut_p = jnp.pad(W_nut, ((0, HD - 20), (0, 0)))
    nutrient_x, _, _ = _mm_call(nut_p, Wnut_p, b_nut, [], NINT)

    z1 = _enc_call(nutrient_x, W1, b1, W2, b2, NINT)
    z2 = _enc_call(caption_x, W1, b1, W2, b2, NIMG)
    loss = _flash_call(z1, z2, NINT)
    csig = jax.nn.sigmoid(loss)

    hi, sc_hi, _ = _mm_call(ingredient_x, Wp_ing, jnp.zeros(HD, _f32), [a1s],
                            NING)
    ht, sc_ht, _ = _mm_call(cooking_x, Wp_taste, jnp.zeros(HD, _f32), [a1d],
                            NT)
    hu, sc_hu, _ = _mm_call(ux, Wn_user, jnp.zeros(HD, _f32),
                            [a_ub_s, a_ib_d], NU, renorm=True)
    hit, sc_hit, _ = _mm_call(_pad_rows(item_x, P10), Wn_item,
                              jnp.zeros(HD, _f32),
                              [a_ti_d, a_ii_d, a_mi_d, a_ub_d, a_ib_s], NIT)
    hg, _, st_hg = _mm_call(ingredient_x, Wn_ing, jnp.zeros(HD, _f32), [],
                            NING)
    hin, sc_hin, st_hin = _mm_call(z1, Wn_int * csig, jnp.zeros(HD, _f32),
                                   [a_ii_s], NINT)
    him, sc_him, st_him = _mm_call(visual_x, Wn_img, jnp.zeros(HD, _f32),
                                   [a_mi_s], NIMG)

    rsh = lambda ei: (ei[0].astype(_i32).reshape(NW, ECH, EC),
                      ei[1].astype(_i32).reshape(NW, ECH, EC))

    sp, dp = rsh(ei_part_of)
    pt = _sc_edge_att(sp, dp, hi, sc_hi[:, 0], sc_ht[:, 0])
    taste_x, _ = _edge_att_combine([pt], NT)

    hta, sc_hta, st_hta = _mm_call(taste_x, Wn_taste, jnp.zeros(HD, _f32),
                                   [a_ti_s], NT)

    s1_, d1_ = rsh(ei_taste_item)
    s2_, d2_ = rsh(ei_intention_item)
    s3_, d3_ = rsh(ei_image_item)
    s4_, d4_ = rsh(ei_user_buys)
    s5_, d5_ = rsh(ei_item_bought)
    sm = _sc_att_softmax_multi([
        (s2_, d2_, sc_hin[:, 0], sc_hit[:, 1]),
        (s3_, d3_, sc_him[:, 0], sc_hit[:, 2]),
        (s4_, d4_, sc_hu[:, 0], sc_hit[:, 3]),
        (s5_, d5_, sc_hit[:, 4], sc_hu[:, 1]),
    ])
    [(e1, S1, M1)] = _sc_att_softmax_multi(
        [(s1_, d1_, sc_hta[:, 0], sc_hit[:, 0])])
    Ps = _sc_att_aggregate_multi([
        (s1_, d1_, e1, hta),
        (s2_, d2_, sm[0][0], hin),
        (s3_, d3_, sm[1][0], him),
        (s4_, d4_, sm[2][0], hu),
        (s5_, d5_, sm[3][0], hit),
    ])
    item_pre, st_item = _edge_att_combine(
        [(Ps[0], S1, M1), (Ps[1], sm[0][1], sm[0][2]),
         (Ps[2], sm[1][1], sm[1][2]), (Ps[3], sm[2][1], sm[2][2])], NIT)
    user_pre, st_user = _edge_att_combine([(Ps[4], sm[3][1], sm[3][2])], NU)

    pieces = []
    for x, st, n in ((user_pre, st_user, NU), (item_pre, st_item, NIT),
                     (hg, st_hg, NING), (hta, st_hta, NT),
                     (hin, st_hin, NINT), (him, st_him, NIMG)):
        sc_, sh_ = _bn_from_stats(st, n, gamma, beta)
        pieces.append(_bn_call(x, sc_, sh_)[:n])
    return jnp.concatenate(pieces, axis=0)


# speedup vs baseline: 1.0315x; 1.0315x over previous
import functools

import jax
import jax.numpy as jnp
from jax import lax
from jax.experimental import pallas as pl
from jax.experimental.pallas import tpu as pltpu
from jax.experimental.pallas import tpu_sc as plsc

HD = 128
NC, NS, L = 2, 16, 16
NW = NC * NS
E = 64000
EC = 80
ECH_TOT = E // EC
ECH = ECH_TOT // NW
ET = E // NW
P10 = 10240
P5 = 5120
BM = 512

_f32 = jnp.float32
_i32 = jnp.int32


def _pad_rows(x, n):
    return jnp.pad(x, ((0, n - x.shape[0]),) + ((0, 0),) * (x.ndim - 1))


def _sc_gather_call(tables, idxs):
    mesh = plsc.VectorSubcoreMesh(core_axis_name="c", subcore_axis_name="s", num_cores=NC, num_subcores=NS)
    specs = [(t.shape, int(i.shape[0])) for t, i in zip(tables, idxs)]
    maxc = max(b // (NW * EC) for _, b in specs)

    out_type = [jax.ShapeDtypeStruct((b, HD), _f32) for _, b in specs]
    nch_tot = sum(b // (NW * EC) for _, b in specs)
    del maxc
    scratch = [pltpu.VMEM((nch_tot, EC), _i32),
               pltpu.VMEM((2, EC, HD), _f32),
               pltpu.SemaphoreType.DMA, pltpu.SemaphoreType.DMA,
               pltpu.SemaphoreType.DMA]

    chunks = []
    for g, (_, b) in enumerate(specs):
        for k in range(b // (NW * EC)):
            chunks.append((g, k))
    ncht = len(chunks)

    def body(*refs):
        n = len(specs)
        tabs = refs[:n]
        idr = refs[n:2 * n]
        outs = refs[2 * n:3 * n]
        idx_v, rows_v, si, sg, so = refs[3 * n:]
        wid = lax.axis_index("c") * NS + lax.axis_index("s")

        def off_of(g, k):
            nch = specs[g][1] // (NW * EC)
            return wid * nch * EC + k * EC

        idd = []
        for i, (g, k) in enumerate(chunks):
            idd.append(pltpu.async_copy(
                idr[g].at[pl.ds(off_of(g, k), EC)], idx_v.at[i], si))
        for d in idd:
            d.wait()
        gd = [None] * ncht
        od = [None] * ncht
        g0, k0 = chunks[0]
        gd[0] = pltpu.async_copy(tabs[g0].at[idx_v.at[0]], rows_v.at[0], sg)
        for i, (g, k) in enumerate(chunks):
            b = i % 2
            gd[i].wait()
            if i + 1 < ncht:
                if i >= 1:
                    od[i - 1].wait()
                g1, k1 = chunks[i + 1]
                gd[i + 1] = pltpu.async_copy(
                    tabs[g1].at[idx_v.at[i + 1]], rows_v.at[1 - b], sg)
            od[i] = pltpu.async_copy(
                rows_v.at[b], outs[g].at[pl.ds(off_of(g, k), EC)], so)
        od[ncht - 2].wait()
        od[ncht - 1].wait()

    fn = pl.kernel(body, out_type=out_type, mesh=mesh, scratch_types=scratch, compiler_params=pltpu.CompilerParams(needs_layout_passes=False))
    return fn(*tables, *idxs)


def _sc_att_softmax_multi(rels):
    nrel = len(rels)
    n_src_max = max(r[2].shape[0] for r in rels)
    mesh = plsc.VectorSubcoreMesh(core_axis_name="c", subcore_axis_name="s", num_cores=NC, num_subcores=NS)
    out_type = ([jax.ShapeDtypeStruct((NW, ECH, EC), _f32)] * nrel +
                [jax.ShapeDtypeStruct((NC, P10 // L, L), _f32)] * nrel +
                [jax.ShapeDtypeStruct((NC, L), _f32)] * nrel)
    scratch = [
        pltpu.VMEM((ECH, EC), _i32),
        pltpu.VMEM((ECH, EC), _i32),
        pltpu.VMEM((n_src_max,), _f32),
        pltpu.VMEM((P10,), _f32),
        pltpu.VMEM((ECH, EC), _f32),
        pltpu.VMEM((2, EC, L), _f32),
        pltpu.VMEM((2, EC), _i32),
        pltpu.VMEM((P10 // L // NS, L), _f32),
        pltpu.VMEM((NS, L), _f32),
        pltpu.VMEM((L,), _f32),
        pltpu.VMEM_SHARED((P10 // L, L), _f32),
        pltpu.VMEM_SHARED((NS, L), _f32),
        pltpu.SemaphoreType.DMA,
    ]

    def body(*refs):
        ins = refs[:4 * nrel]
        outE = refs[4 * nrel:5 * nrel]
        outS = refs[5 * nrel:6 * nrel]
        outM = refs[6 * nrel:7 * nrel]
        (src2, dst2, ssrc_v, sdst_v, abuf, oh, ohx, sbuf,
         mall, mbuf, S_sh2, M_sh, so) = refs[7 * nrel:]
        cid = lax.axis_index("c")
        sid = lax.axis_index("s")
        wid = cid * NS + sid
        zv = jnp.zeros((L,), _f32)
        nsr = P10 // L // NS
        iv = lax.iota(_i32, L)

        def zsb(i, _):
            sbuf[i, :] = zv
            return 0
        lax.fori_loop(0, nsr, zsb, 0)

        for rel in range(nrel):
            src_h, dst_h, ssrc_h, sdst_h = ins[4 * rel:4 * rel + 4]
            nsp = ssrc_h.shape[0]

            pltpu.sync_copy(sbuf, S_sh2.at[pl.ds(sid * nsr, nsr)])

            pltpu.sync_copy(src_h.at[wid], src2)
            pltpu.sync_copy(dst_h.at[wid], dst2)
            pltpu.sync_copy(ssrc_h, ssrc_v.at[pl.ds(0, nsp)])
            pltpu.sync_copy(sdst_h, sdst_v)

            def apass(ch, mv):
                for v in range(EC // L):
                    s16 = src2[ch, pl.ds(v * L, L)]
                    d16 = dst2[ch, pl.ds(v * L, L)]
                    a = plsc.load_gather(ssrc_v, [s16])
                    b = plsc.load_gather(sdst_v, [d16])
                    al = a + b
                    al = jnp.where(al > 0, al, 0.2 * al)
                    abuf[ch, pl.ds(v * L, L)] = al
                    mv = jnp.maximum(mv, al)
                return mv
            mv = lax.fori_loop(0, ECH, apass, jnp.full((L,), -1e30, _f32))

            mbuf[...] = mv
            pltpu.sync_copy(mbuf, M_sh.at[sid])
            plsc.subcore_barrier()
            pltpu.sync_copy(M_sh, mall)
            mm = mall[0, :]
            for i in range(1, NS):
                mm = jnp.maximum(mm, mall[i, :])
            M = jnp.max(mm)

            sdd = [None] * ECH
            for ch in range(ECH):
                b = ch % 2
                if ch >= 2:
                    sdd[ch - 2].wait()

                def zoh(r, _2, b=b):
                    oh[b, r, :] = zv
                    return 0
                lax.fori_loop(0, EC, zoh, 0)
                for v in range(EC // L):
                    d16 = dst2[ch, pl.ds(v * L, L)]
                    al = abuf[ch, pl.ds(v * L, L)]
                    e = jnp.exp(al - M)
                    abuf[ch, pl.ds(v * L, L)] = e
                    plsc.store_scatter(
                        oh.at[b], [iv + v * L,
                                   jnp.bitwise_and(d16, L - 1)], e)
                    ohx[b, pl.ds(v * L, L)] = lax.shift_right_logical(d16, 4)
                sdd[ch] = pltpu.async_copy(oh.at[b], S_sh2.at[ohx.at[b]],
                                           so, add=True)
            sdd[ECH - 2].wait()
            sdd[ECH - 1].wait()

            pltpu.sync_copy(abuf, outE[rel].at[wid])
            plsc.subcore_barrier()

            pltpu.sync_copy(S_sh2.at[pl.ds(sid * nsr, nsr)],
                            outS[rel].at[cid, pl.ds(sid * nsr, nsr)])

            mbuf[...] = jnp.full((L,), M, _f32)

            @pl.when(sid == 0)
            def _():
                pltpu.sync_copy(mbuf, outM[rel].at[cid])

    fn = pl.kernel(body, out_type=out_type, mesh=mesh, scratch_types=scratch, compiler_params=pltpu.CompilerParams(needs_layout_passes=False))
    outs = fn(*[x for r in rels for x in r])
    return [(outs[i], outs[nrel + i], outs[2 * nrel + i])
            for i in range(nrel)]


def _sc_att_aggregate_multi(rels):
    nrel = len(rels)
    mesh = plsc.VectorSubcoreMesh(core_axis_name="c", subcore_axis_name="s", num_cores=NC, num_subcores=NS)
    out_type = [jax.ShapeDtypeStruct((NC, P10, HD), _f32)] * nrel
    scratch = [
        pltpu.VMEM((ECH, EC), _i32),
        pltpu.VMEM((ECH, EC), _i32),
        pltpu.VMEM((ECH, EC), _f32),
        pltpu.VMEM((2, EC, HD), _f32),
        pltpu.VMEM_SHARED((P10, HD), _f32),
        pltpu.SemaphoreType.DMA,
        pltpu.SemaphoreType.DMA,
    ]

    def body(*refs):
        ins = refs[:4 * nrel]
        outP = refs[4 * nrel:5 * nrel]
        src2, dst2, ebuf_v, rows, P_sh, sg, ss = refs[5 * nrel:]
        cid = lax.axis_index("c")
        sid = lax.axis_index("s")
        wid = cid * NS + sid
        zv = jnp.zeros((L,), _f32)
        nrow_t = P10 // NS

        for rel in range(nrel):
            src_h, dst_h, e_h, h_h = ins[4 * rel:4 * rel + 4]

            def zrow(i, _):
                for j in range(HD // L):
                    rows[0, i, pl.ds(j * L, L)] = zv
                return 0
            lax.fori_loop(0, EC, zrow, 0)
            for k in range(nrow_t // EC):
                pltpu.sync_copy(rows.at[0],
                                P_sh.at[pl.ds(sid * nrow_t + k * EC, EC)])

            pltpu.sync_copy(src_h.at[wid], src2)
            pltpu.sync_copy(dst_h.at[wid], dst2)
            pltpu.sync_copy(e_h.at[wid], ebuf_v)

            plsc.subcore_barrier()

            gd = [None] * ECH
            sd = [None] * ECH
            gd[0] = pltpu.async_copy(h_h.at[src2.at[0]], rows.at[0], sg)
            for ch in range(ECH):
                b = ch % 2
                gd[ch].wait()
                if ch >= 1:
                    sd[ch - 1].wait()
                if ch + 1 < ECH:
                    gd[ch + 1] = pltpu.async_copy(
                        h_h.at[src2.at[ch + 1]], rows.at[1 - b], sg)

                def scale(r, _2, ch=ch, b=b):
                    ev = plsc.load_gather(
                        ebuf_v,
                        [jnp.full((L,), ch, _i32), jnp.full((L,), r, _i32)])
                    for j in range(HD // L):
                        rows[b, r, pl.ds(j * L, L)] = (
                            rows[b, r, pl.ds(j * L, L)] * ev)
                    return 0
                lax.fori_loop(0, EC, scale, 0)
                sd[ch] = pltpu.async_copy(rows.at[b], P_sh.at[dst2.at[ch]],
                                          ss, add=True)
            sd[ECH - 1].wait()

            plsc.subcore_barrier()
            pltpu.sync_copy(P_sh.at[pl.ds(sid * nrow_t, nrow_t)],
                            outP[rel].at[cid, pl.ds(sid * nrow_t, nrow_t)])

    fn = pl.kernel(body, out_type=out_type, mesh=mesh, scratch_types=scratch, compiler_params=pltpu.CompilerParams(needs_layout_passes=False))
    outs = fn(*[x for r in rels for x in r])
    return list(outs) if nrel > 1 else [outs[0] if isinstance(outs, (list, tuple)) else outs]


def _sc_edge_att(src_r, dst_r, h_src, s_src, s_dst):
    [(ebuf, S, M)] = _sc_att_softmax_multi([(src_r, dst_r, s_src, s_dst)])
    [P] = _sc_att_aggregate_multi([(src_r, dst_r, ebuf, h_src)])
    return P, S, M


def _edge_att_combine(parts, n_dst):
    ws = []
    for (Pp, Sr, Mm) in parts:
        m0 = Mm[0, 0]
        m1 = Mm[1, 0]
        mg = jnp.maximum(m0, m1)
        ws.append(jnp.stack([jnp.exp(m0 - mg), jnp.exp(m1 - mg)]))
    w = jnp.concatenate(ws)
    arrs = []
    for (Pp, Sr, Mm) in parts:
        arrs.extend([Pp, Sr.reshape(NC, P10 // HD, HD)])
    return _comb_call(arrs, w, len(parts))


def _mm_body(xr, wr, br, ar, yr, scr, str_, *, n_real, renorm):
    i = pl.program_id(0)
    x = xr[...]
    if renorm:
        nrm = jnp.sqrt(jnp.sum(x * x, axis=1, keepdims=True))
        x = x * jnp.minimum(1.0, 1.0 / (nrm + 1e-12))
    y = jnp.dot(x, wr[...], preferred_element_type=_f32) + br[...]
    rid = lax.broadcasted_iota(_i32, y.shape, 0) + i * BM
    y = jnp.where(rid < n_real, y, 0.0)
    yr[...] = y
    scr[...] = jnp.dot(y, ar[...], preferred_element_type=_f32)
    s1 = jnp.sum(y, axis=0, keepdims=True)
    s2 = jnp.sum(y * y, axis=0, keepdims=True)
    str_[...] = jnp.concatenate([s1, s2], axis=0)[None]


def _mm_call(x, W, b, avecs, n_real, renorm=False):
    np_ = x.shape[0]
    nblk = np_ // BM
    A = jnp.zeros((HD, 8), _f32)
    for k, v in enumerate(avecs):
        A = A.at[:, k].set(v)
    f = pl.pallas_call(
        functools.partial(_mm_body, n_real=n_real, renorm=renorm),
        grid=(nblk,),
        in_specs=[pl.BlockSpec((BM, HD), lambda i: (i, 0)),
                  pl.BlockSpec((HD, HD), lambda i: (0, 0)),
                  pl.BlockSpec((1, HD), lambda i: (0, 0)),
                  pl.BlockSpec((HD, 8), lambda i: (0, 0))],
        out_specs=[pl.BlockSpec((BM, HD), lambda i: (i, 0)),
                   pl.BlockSpec((BM, 8), lambda i: (i, 0)),
                   pl.BlockSpec((1, 2, HD), lambda i: (i, 0, 0))],
        out_shape=[jax.ShapeDtypeStruct((np_, HD), _f32),
                   jax.ShapeDtypeStruct((np_, 8), _f32),
                   jax.ShapeDtypeStruct((nblk, 2, HD), _f32)],
    )
    return f(x, W, b.reshape(1, HD), A)


def _enc_body(xr, w1r, b1r, w2r, b2r, zr, *, n_real):
    i = pl.program_id(0)
    h = jnp.maximum(jnp.dot(xr[...], w1r[...], preferred_element_type=_f32)
                    + b1r[...], 0.0)
    z = jnp.dot(h, w2r[...], preferred_element_type=_f32) + b2r[...]
    nrm = jnp.sqrt(jnp.sum(z * z, axis=1, keepdims=True))
    z = z / (nrm + 1e-12)
    rid = lax.broadcasted_iota(_i32, z.shape, 0) + i * BM
    zr[...] = jnp.where(rid < n_real, z, 0.0)


def _enc_call(x, W1, b1, W2, b2, n_real):
    np_ = x.shape[0]
    f = pl.pallas_call(
        functools.partial(_enc_body, n_real=n_real),
        grid=(np_ // BM,),
        in_specs=[pl.BlockSpec((BM, HD), lambda i: (i, 0)),
                  pl.BlockSpec((HD, HD), lambda i: (0, 0)),
                  pl.BlockSpec((1, HD), lambda i: (0, 0)),
                  pl.BlockSpec((HD, HD), lambda i: (0, 0)),
                  pl.BlockSpec((1, HD), lambda i: (0, 0))],
        out_specs=pl.BlockSpec((BM, HD), lambda i: (i, 0)),
        out_shape=jax.ShapeDtypeStruct((np_, HD), _f32),
    )
    return f(x, W1, b1.reshape(1, HD), W2, b2.reshape(1, HD))


def _flash_body(z1r, z2r, outr, *, n_real, np_):
    i = pl.program_id(0)
    z1b = z1r[...]
    CB = 2048

    def col(c, acc):
        z2c = z2r[pl.ds(c * CB, CB), :]
        s = lax.dot_general(z1b, z2c, (((1,), (1,)), ((), ()))) * 2.0
        cidx = lax.broadcasted_iota(_i32, s.shape, 1) + c * CB
        e = jnp.where(cidx < n_real, jnp.exp(s), 0.0)
        return acc + jnp.sum(e, axis=1, keepdims=True)
    acc = lax.fori_loop(0, np_ // CB, col, jnp.zeros((BM, 1), _f32))

    z2b = z2r[pl.ds(i * BM, BM), :]
    dg = jnp.sum(z1b * z2b, axis=1, keepdims=True) * 2.0
    rid = lax.broadcasted_iota(_i32, (BM, 1), 0) + i * BM
    contrib = jnp.where(rid < n_real, jnp.log(acc) - dg, 0.0)

    @pl.when(i == 0)
    def _():
        outr[...] = jnp.zeros((1, 1), _f32)
    outr[...] += jnp.sum(contrib).reshape(1, 1)


def _flash_call(z1, z2, n_real):
    np_ = z1.shape[0]
    f = pl.pallas_call(
        functools.partial(_flash_body, n_real=n_real, np_=np_),
        grid=(np_ // BM,),
        in_specs=[pl.BlockSpec((BM, HD), lambda i: (i, 0)),
                  pl.BlockSpec((np_, HD), lambda i: (0, 0))],
        out_specs=pl.BlockSpec((1, 1), lambda i: (0, 0)),
        out_shape=jax.ShapeDtypeStruct((1, 1), _f32),
    )
    return f(z1, z2)[0, 0] / n_real


def _comb_body(*refs, nrel, scale):
    wr = refs[0]
    yr, str_ = refs[-2], refs[-1]
    acc = None
    i = pl.program_id(0)
    nsb = BM // HD
    lane = lax.broadcasted_iota(_i32, (BM, HD), 1)
    rowm = jnp.bitwise_and(lax.broadcasted_iota(_i32, (BM, HD), 0), HD - 1)
    emask = jnp.where(lane == rowm, 1.0, 0.0)
    for r in range(nrel):
        Pr = refs[1 + 2 * r][...]
        Sr = refs[2 + 2 * r][:, pl.ds(i * nsb, nsb), :]
        num = wr[2 * r] * Pr[0] + wr[2 * r + 1] * Pr[1]
        den = wr[2 * r] * Sr[0] + wr[2 * r + 1] * Sr[1] + 1e-16
        dfull = jnp.reshape(
            lax.broadcast_in_dim(den, (nsb, HD, HD), (0, 2)), (BM, HD))
        dcol = jnp.sum(dfull * emask, axis=1, keepdims=True)
        t = num / dcol
        acc = t if acc is None else acc + t
    y = jnp.maximum(acc * scale, 0.0)
    yr[...] = y
    s1 = jnp.sum(y, axis=0, keepdims=True)
    s2 = jnp.sum(y * y, axis=0, keepdims=True)
    str_[...] = jnp.concatenate([s1, s2], axis=0)[None]


def _comb_call(arrs, w, nrel):
    nblk = P10 // BM
    scale = 1.0 / nrel
    specs = []
    for k in range(2 * nrel):
        if k % 2 == 0:
            specs.append(pl.BlockSpec((NC, BM, HD), lambda i: (0, i, 0)))
        else:
            specs.append(pl.BlockSpec((NC, P10 // HD, HD),
                                      lambda i: (0, 0, 0)))
    f = pl.pallas_call(
        functools.partial(_comb_body, nrel=nrel, scale=scale),
        grid=(nblk,),
        in_specs=[pl.BlockSpec(memory_space=pltpu.SMEM)] + specs,
        out_specs=[pl.BlockSpec((BM, HD), lambda i: (i, 0)),
                   pl.BlockSpec((1, 2, HD), lambda i: (i, 0, 0))],
        out_shape=[jax.ShapeDtypeStruct((P10, HD), _f32),
                   jax.ShapeDtypeStruct((nblk, 2, HD), _f32)],
    )
    return f(w, *arrs)


def _bn_body(xr, scr, shr, yr):
    yr[...] = xr[...] * scr[...] + shr[...]


def _bn_call(x, sc, sh):
    np_ = x.shape[0]
    f = pl.pallas_call(
        _bn_body,
        grid=(np_ // BM,),
        in_specs=[pl.BlockSpec((BM, HD), lambda i: (i, 0)),
                  pl.BlockSpec((1, HD), lambda i: (0, 0)),
                  pl.BlockSpec((1, HD), lambda i: (0, 0))],
        out_specs=pl.BlockSpec((BM, HD), lambda i: (i, 0)),
        out_shape=jax.ShapeDtypeStruct((np_, HD), _f32),
    )
    return f(x, sc.reshape(1, HD), sh.reshape(1, HD))


def _bn_from_stats(st, n_real, gamma, beta):
    s = jnp.sum(st, axis=0)
    mean = s[0] / n_real
    var = s[1] / n_real - mean * mean
    scale = gamma / jnp.sqrt(var + 1e-5)
    shift = beta - mean * scale
    return scale, shift


def kernel(user_ids, image_ids, ingredient_ids, taste_ids, nutrient, item_x,
           ei_part_of, ei_taste_item, ei_intention_item, ei_image_item,
           ei_user_buys, ei_item_bought, user_table, visual_table,
           caption_table, ing_table, direction_table, W_nut, b_nut, W1, b1,
           W2, b2, Wp_ing, Wp_taste, a1s, a1d, Wn_user, Wn_item, Wn_ing,
           Wn_taste, Wn_int, Wn_img, a_ti_s, a_ti_d, a_ii_s, a_ii_d, a_mi_s,
           a_mi_d, a_ub_s, a_ub_d, a_ib_s, a_ib_d, gamma, beta):
    NU, NIT, NING, NT = 10000, 10000, 5000, 10000
    NINT, NIMG = 10000, 10000

    pad_i = lambda ids, n: jnp.pad(ids.astype(_i32), (0, n - ids.shape[0]))
    ux, visual_x, caption_x, ingredient_x, cooking_x = _sc_gather_call(
        [user_table, visual_table, caption_table, ing_table, direction_table],
        [pad_i(user_ids, P10), pad_i(image_ids, P10), pad_i(image_ids, P10),
         pad_i(ingredient_ids, P5), pad_i(taste_ids, P10)])

    nut_p = _pad_rows(jnp.pad(nutrient, ((0, 0), (0, HD - 20))), P10)
    Wnut_p = jnp.pad(W_nut, ((0, HD - 20), (0, 0)))
    nutrient_x, _, _ = _mm_call(nut_p, Wnut_p, b_nut, [], NINT)

    z1 = _enc_call(nutrient_x, W1, b1, W2, b2, NINT)
    z2 = _enc_call(caption_x, W1, b1, W2, b2, NIMG)
    loss = _flash_call(z1, z2, NINT)
    csig = jax.nn.sigmoid(loss)

    hi, sc_hi, _ = _mm_call(ingredient_x, Wp_ing, jnp.zeros(HD, _f32), [a1s],
                            NING)
    ht, sc_ht, _ = _mm_call(cooking_x, Wp_taste, jnp.zeros(HD, _f32), [a1d],
                            NT)
    hu, sc_hu, _ = _mm_call(ux, Wn_user, jnp.zeros(HD, _f32),
                            [a_ub_s, a_ib_d], NU, renorm=True)
    hit, sc_hit, _ = _mm_call(_pad_rows(item_x, P10), Wn_item,
                              jnp.zeros(HD, _f32),
                              [a_ti_d, a_ii_d, a_mi_d, a_ub_d, a_ib_s], NIT)
    hg, _, st_hg = _mm_call(ingredient_x, Wn_ing, jnp.zeros(HD, _f32), [],
                            NING)
    hin, sc_hin, st_hin = _mm_call(z1, Wn_int * csig, jnp.zeros(HD, _f32),
                                   [a_ii_s], NINT)
    him, sc_him, st_him = _mm_call(visual_x, Wn_img, jnp.zeros(HD, _f32),
                                   [a_mi_s], NIMG)

    rsh = lambda ei: (ei[0].astype(_i32).reshape(NW, ECH, EC),
                      ei[1].astype(_i32).reshape(NW, ECH, EC))

    sp, dp = rsh(ei_part_of)
    pt = _sc_edge_att(sp, dp, hi, sc_hi[:, 0], sc_ht[:, 0])
    taste_x, _ = _edge_att_combine([pt], NT)

    hta, sc_hta, st_hta = _mm_call(taste_x, Wn_taste, jnp.zeros(HD, _f32),
                                   [a_ti_s], NT)

    s1_, d1_ = rsh(ei_taste_item)
    p1 = _sc_edge_att(s1_, d1_, hta, sc_hta[:, 0], sc_hit[:, 0])
    s2_, d2_ = rsh(ei_intention_item)
    p2 = _sc_edge_att(s2_, d2_, hin, sc_hin[:, 0], sc_hit[:, 1])
    s3_, d3_ = rsh(ei_image_item)
    p3 = _sc_edge_att(s3_, d3_, him, sc_him[:, 0], sc_hit[:, 2])
    s4_, d4_ = rsh(ei_user_buys)
    p4 = _sc_edge_att(s4_, d4_, hu, sc_hu[:, 0], sc_hit[:, 3])
    item_pre, st_item = _edge_att_combine([p1, p2, p3, p4], NIT)

    s5_, d5_ = rsh(ei_item_bought)
    p5 = _sc_edge_att(s5_, d5_, hit, sc_hit[:, 4], sc_hu[:, 1])
    user_pre, st_user = _edge_att_combine([p5], NU)

    pieces = []
    for x, st, n in ((user_pre, st_user, NU), (item_pre, st_item, NIT),
                     (hg, st_hg, NING), (hta, st_hta, NT),
                     (hin, st_hin, NINT), (him, st_him, NIMG)):
        sc_, sh_ = _bn_from_stats(st, n, gamma, beta)
        pieces.append(_bn_call(x, sc_, sh_)[:n])
    return jnp.concatenate(pieces, axis=0)
